# baseline (speedup 1.0000x reference)
import functools

import jax
import jax.numpy as jnp
from jax import lax
from jax.experimental import pallas as pl
from jax.experimental.pallas import tpu as pltpu
from jax.experimental.pallas import tpu_sc as plsc

_B = 16384
_D = 64
_L = 16
_R = 1000

_info = plsc.get_sparse_core_info()
_NC, _NS = _info.num_cores, _info.num_subcores
_NW = _NC * _NS
_BPW = _B // _NW
_GROUPS = _BPW // _L


def _fetch_group(ent, rel, hidx, ridx, tidx, hblk, rblk, tblk, sem, g, par):
    hvec = hidx[pl.ds(g * _L, _L)] >> 3
    rvec = ridx[pl.ds(g * _L, _L)] >> 3
    tvec = tidx[pl.ds(g * _L, _L)] >> 3
    for j in range(_L):
        pltpu.async_copy(ent.at[hvec[j]], hblk.at[par, j], sem)
        pltpu.async_copy(rel.at[rvec[j]], rblk.at[par, j], sem)
        pltpu.async_copy(ent.at[tvec[j]], tblk.at[par, j], sem)


def _drain_group(ent, hblk, rblk, tblk, sem, par):
    for buf in (hblk, rblk, tblk):
        pltpu.make_async_copy(ent.at[pl.ds(0, _L)], buf.at[par], sem).wait()


def _sc_body(heads_hbm, rels_hbm, tails_hbm, ent, rel, out_hbm,
             hidx, ridx, tidx, hblk, rblk, tblk, outv, sem):
    wid = lax.axis_index("s") * _NC + lax.axis_index("c")
    base = wid * _BPW

    pltpu.sync_copy(heads_hbm.at[pl.ds(base, _BPW)], hidx)
    pltpu.sync_copy(rels_hbm.at[pl.ds(base, _BPW)], ridx)
    pltpu.sync_copy(tails_hbm.at[pl.ds(base, _BPW)], tidx)

    _fetch_group(ent, rel, hidx, ridx, tidx, hblk, rblk, tblk, sem, 0, 0)

    lane = lax.iota(jnp.int32, _L)
    perms = [lane ^ d for d in (1, 2, 4, 8)]
    masks = [(lane & d) != 0 for d in (1, 2, 4, 8)]

    dnums = lax.GatherDimensionNumbers(
        offset_dims=(), collapsed_slice_dims=(0,), start_index_map=(0,))

    def vperm(x, idx):
        return lax.gather(x, idx[:, None], dnums, (1,),
                          mode=lax.GatherScatterMode.PROMISE_IN_BOUNDS)

    def merge(a, b, lvl):
        pa = vperm(a, perms[lvl])
        pb = vperm(b, perms[lvl])
        return jnp.where(masks[lvl], b + pb, a + pa)

    def group_body(g, carry):
        par = lax.rem(g, 2)

        @pl.when(g + 1 < _GROUPS)
        def _():
            _fetch_group(ent, rel, hidx, ridx, tidx, hblk, rblk, tblk,
                         sem, g + 1, 1 - par)

        _drain_group(ent, hblk, rblk, tblk, sem, par)

        hodd = hidx[pl.ds(g * _L, _L)] & 1
        rodd = ridx[pl.ds(g * _L, _L)] & 1
        todd = tidx[pl.ds(g * _L, _L)] & 1

        ps = []
        for j in range(_L):
            hm = jnp.full((_L,), hodd[j], jnp.int32).astype(jnp.float32)
            rm = jnp.full((_L,), rodd[j], jnp.int32).astype(jnp.float32)
            tm = jnp.full((_L,), todd[j], jnp.int32).astype(jnp.float32)
            acc = None
            for c in range(_D // _L):
                lo = pl.ds(c * _L, _L)
                hi = pl.ds(_D + c * _L, _L)
                hv = hblk[par, j, lo] + hm * (hblk[par, j, hi]
                                              - hblk[par, j, lo])
                rv = rblk[par, j, lo] + rm * (rblk[par, j, hi]
                                              - rblk[par, j, lo])
                tv = tblk[par, j, lo] + tm * (tblk[par, j, hi]
                                              - tblk[par, j, lo])
                df = hv + rv - tv
                sq = df * df
                acc = sq if acc is None else acc + sq
            ps.append(acc)
        for lvl in range(4):
            ps = [merge(ps[2 * j], ps[2 * j + 1], lvl)
                  for j in range(len(ps) // 2)]
        acc = ps[0]

        a = jnp.maximum(acc, jnp.float32(1e-30))
        i = lax.bitcast_convert_type(a, jnp.int32)
        i = jnp.int32(0x5F3759DF) - lax.shift_right_logical(
            i, jnp.ones((_L,), jnp.int32))
        y = lax.bitcast_convert_type(i, jnp.float32)
        half_a = jnp.float32(0.5) * a
        for _ in range(3):
            y = y * (jnp.float32(1.5) - half_a * y * y)
        outv[pl.ds(g * _L, _L)] = a * y
        return carry

    lax.fori_loop(0, _GROUPS, group_body, jnp.int32(0))
    pltpu.sync_copy(outv, out_hbm.at[pl.ds(base, _BPW)])


@functools.partial(
    pl.kernel,
    mesh=plsc.VectorSubcoreMesh(core_axis_name="c", subcore_axis_name="s"),
    out_type=jax.ShapeDtypeStruct((_B,), jnp.float32),
    compiler_params=pltpu.CompilerParams(use_tc_tiling_on_sc=True),
    scratch_types=[
        pltpu.VMEM((_BPW,), jnp.int32),
        pltpu.VMEM((_BPW,), jnp.int32),
        pltpu.VMEM((_BPW,), jnp.int32),
        pltpu.VMEM((2, _L, 2 * _D), jnp.float32),
        pltpu.VMEM((2, _L, 2 * _D), jnp.float32),
        pltpu.VMEM((2, _L, 2 * _D), jnp.float32),
        pltpu.VMEM((_BPW,), jnp.float32),
        pltpu.SemaphoreType.DMA,
    ],
)
def _transe_sc(heads_hbm, rels_hbm, tails_hbm, ent, rel, out_hbm,
               hidx, ridx, tidx, hblk, rblk, tblk, outv, sem):
    _sc_body(heads_hbm, rels_hbm, tails_hbm, ent, rel, out_hbm,
             hidx, ridx, tidx, hblk, rblk, tblk, outv, sem)


def kernel(heads, relations, tails, entity_emb, relation_emb):
    h32 = heads.astype(jnp.int32)
    r32 = relations.astype(jnp.int32)
    t32 = tails.astype(jnp.int32)
    ent2 = entity_emb.reshape(500000, 2 * _D)
    rel2 = relation_emb.reshape(500, 2 * _D)
    return _transe_sc(h32, r32, t32, ent2, rel2)


# speedup vs baseline: 2.0150x; 2.0150x over previous
import functools

import jax
import jax.numpy as jnp
from jax import lax
from jax.experimental import pallas as pl
from jax.experimental.pallas import tpu as pltpu
from jax.experimental.pallas import tpu_sc as plsc

_B = 16384
_D = 64
_L = 16
_R = 1000

_info = plsc.get_sparse_core_info()
_NC, _NS = _info.num_cores, _info.num_subcores
_NW = _NC * _NS
_BPW = _B // _NW
_GROUPS = _BPW // _L


def _fetch_group(ent, rel, hidx, ridx, tidx, hblk, rblk, tblk, sem, g, par):
    hvec = hidx[pl.ds(g * _L, _L)] >> 3
    rvec = ridx[pl.ds(g * _L, _L)] >> 3
    tvec = tidx[pl.ds(g * _L, _L)] >> 3
    for j in range(_L):
        pltpu.async_copy(ent.at[hvec[j]], hblk.at[par, j], sem)
        pltpu.async_copy(rel.at[rvec[j]], rblk.at[par, j], sem)
        pltpu.async_copy(ent.at[tvec[j]], tblk.at[par, j], sem)


def _drain_group(ent, hblk, rblk, tblk, sem, par):
    for buf in (hblk, rblk, tblk):
        pltpu.make_async_copy(ent.at[pl.ds(0, _L)], buf.at[par], sem).wait()


def _sc_body(heads_hbm, rels_hbm, tails_hbm, ent, rel, out_hbm,
             hidx, ridx, tidx, hblk, rblk, tblk, outv, sem):
    wid = lax.axis_index("s") * _NC + lax.axis_index("c")
    base = wid * _BPW

    pltpu.sync_copy(heads_hbm.at[pl.ds(base, _BPW)], hidx)
    pltpu.sync_copy(rels_hbm.at[pl.ds(base, _BPW)], ridx)
    pltpu.sync_copy(tails_hbm.at[pl.ds(base, _BPW)], tidx)

    _fetch_group(ent, rel, hidx, ridx, tidx, hblk, rblk, tblk, sem, 0, 0)

    lane = lax.iota(jnp.int32, _L)
    perms = [lane ^ d for d in (1, 2, 4, 8)]
    masks = [(lane & d) != 0 for d in (1, 2, 4, 8)]

    dnums = lax.GatherDimensionNumbers(
        offset_dims=(), collapsed_slice_dims=(0,), start_index_map=(0,))

    def vperm(x, idx):
        return lax.gather(x, idx[:, None], dnums, (1,),
                          mode=lax.GatherScatterMode.PROMISE_IN_BOUNDS)

    def merge(a, b, lvl):
        pa = vperm(a, perms[lvl])
        pb = vperm(b, perms[lvl])
        return jnp.where(masks[lvl], b + pb, a + pa)

    def group_body(g, carry):
        par = lax.rem(g, 2)

        @pl.when(g + 1 < _GROUPS)
        def _():
            _fetch_group(ent, rel, hidx, ridx, tidx, hblk, rblk, tblk,
                         sem, g + 1, 1 - par)

        _drain_group(ent, hblk, rblk, tblk, sem, par)

        hsub = hidx[pl.ds(g * _L, _L)] & 7
        rsub = ridx[pl.ds(g * _L, _L)] & 7
        tsub = tidx[pl.ds(g * _L, _L)] & 7

        ps = []
        for j in range(_L):
            hr = hsub[j]
            rr = rsub[j]
            tr = tsub[j]
            acc = None
            for c in range(_D // _L):
                sl = pl.ds(c * _L, _L)
                df = (hblk[par, j, hr, sl] + rblk[par, j, rr, sl]
                      - tblk[par, j, tr, sl])
                sq = df * df
                acc = sq if acc is None else acc + sq
            ps.append(acc)
        for lvl in range(4):
            ps = [merge(ps[2 * j], ps[2 * j + 1], lvl)
                  for j in range(len(ps) // 2)]
        acc = ps[0]

        a = jnp.maximum(acc, jnp.float32(1e-30))
        i = lax.bitcast_convert_type(a, jnp.int32)
        i = jnp.int32(0x5F3759DF) - lax.shift_right_logical(
            i, jnp.ones((_L,), jnp.int32))
        y = lax.bitcast_convert_type(i, jnp.float32)
        half_a = jnp.float32(0.5) * a
        for _ in range(3):
            y = y * (jnp.float32(1.5) - half_a * y * y)
        outv[pl.ds(g * _L, _L)] = a * y
        return carry

    lax.fori_loop(0, _GROUPS, group_body, jnp.int32(0))
    pltpu.sync_copy(outv, out_hbm.at[pl.ds(base, _BPW)])


@functools.partial(
    pl.kernel,
    mesh=plsc.VectorSubcoreMesh(core_axis_name="c", subcore_axis_name="s"),
    out_type=jax.ShapeDtypeStruct((_B,), jnp.float32),
    compiler_params=pltpu.CompilerParams(use_tc_tiling_on_sc=True),
    scratch_types=[
        pltpu.VMEM((_BPW,), jnp.int32),
        pltpu.VMEM((_BPW,), jnp.int32),
        pltpu.VMEM((_BPW,), jnp.int32),
        pltpu.VMEM((2, _L, 8, _D), jnp.float32),
        pltpu.VMEM((2, _L, 8, _D), jnp.float32),
        pltpu.VMEM((2, _L, 8, _D), jnp.float32),
        pltpu.VMEM((_BPW,), jnp.float32),
        pltpu.SemaphoreType.DMA,
    ],
)
def _transe_sc(heads_hbm, rels_hbm, tails_hbm, ent, rel, out_hbm,
               hidx, ridx, tidx, hblk, rblk, tblk, outv, sem):
    _sc_body(heads_hbm, rels_hbm, tails_hbm, ent, rel, out_hbm,
             hidx, ridx, tidx, hblk, rblk, tblk, outv, sem)


def kernel(heads, relations, tails, entity_emb, relation_emb):
    h32 = heads.astype(jnp.int32)
    r32 = relations.astype(jnp.int32)
    t32 = tails.astype(jnp.int32)
    ent3 = entity_emb.reshape(125000, 8, _D)
    rel3 = relation_emb.reshape(125, 8, _D)
    return _transe_sc(h32, r32, t32, ent3, rel3)


# speedup vs baseline: 2.0176x; 1.0013x over previous
import functools

import jax
import jax.numpy as jnp
from jax import lax
from jax.experimental import pallas as pl
from jax.experimental.pallas import tpu as pltpu
from jax.experimental.pallas import tpu_sc as plsc

_B = 16384
_D = 64
_L = 16

_info = plsc.get_sparse_core_info()
_NC, _NS = _info.num_cores, _info.num_subcores
_NW = _NC * _NS
_BPW = _B // _NW
_GROUPS = _BPW // _L


def _fetch_group(ent, rel, hidx, ridx, tidx, hblk, rblk, tblk, sem, g, par):
    hvec = hidx[pl.ds(g * _L, _L)] >> 3
    rvec = ridx[pl.ds(g * _L, _L)] >> 3
    tvec = tidx[pl.ds(g * _L, _L)] >> 3
    for j in range(_L):
        pltpu.async_copy(ent.at[hvec[j]], hblk.at[par, j], sem)
        pltpu.async_copy(rel.at[rvec[j]], rblk.at[par, j], sem)
        pltpu.async_copy(ent.at[tvec[j]], tblk.at[par, j], sem)


def _drain_group(ent, hblk, rblk, tblk, sem, par):
    for buf in (hblk, rblk, tblk):
        pltpu.make_async_copy(ent.at[pl.ds(0, _L)], buf.at[par], sem).wait()


def _sc_body(heads_hbm, rels_hbm, tails_hbm, ent, rel, out_hbm,
             hidx, ridx, tidx, hblk, rblk, tblk, outv, sem):
    wid = lax.axis_index("s") * _NC + lax.axis_index("c")
    base = wid * _BPW

    pltpu.sync_copy(heads_hbm.at[pl.ds(base, _BPW)], hidx)
    pltpu.sync_copy(rels_hbm.at[pl.ds(base, _BPW)], ridx)
    pltpu.sync_copy(tails_hbm.at[pl.ds(base, _BPW)], tidx)

    _fetch_group(ent, rel, hidx, ridx, tidx, hblk, rblk, tblk, sem, 0, 0)

    lane = lax.iota(jnp.int32, _L)
    perms = [lane ^ d for d in (1, 2, 4, 8)]
    masks = [(lane & d) != 0 for d in (1, 2, 4, 8)]

    dnums = lax.GatherDimensionNumbers(
        offset_dims=(), collapsed_slice_dims=(0,), start_index_map=(0,))

    def vperm(x, idx):
        return lax.gather(x, idx[:, None], dnums, (1,),
                          mode=lax.GatherScatterMode.PROMISE_IN_BOUNDS)

    def merge(a, b, lvl):
        pa = vperm(a, perms[lvl])
        pb = vperm(b, perms[lvl])
        return jnp.where(masks[lvl], b + pb, a + pa)

    def group_body(g, carry):
        par = lax.rem(g, 2)

        @pl.when(g + 1 < _GROUPS)
        def _():
            _fetch_group(ent, rel, hidx, ridx, tidx, hblk, rblk, tblk,
                         sem, g + 1, 1 - par)

        _drain_group(ent, hblk, rblk, tblk, sem, par)

        hsub = hidx[pl.ds(g * _L, _L)] & 7
        rsub = ridx[pl.ds(g * _L, _L)] & 7
        tsub = tidx[pl.ds(g * _L, _L)] & 7

        ps = []
        for j in range(_L):
            hr = hsub[j]
            rr = rsub[j]
            tr = tsub[j]
            acc = None
            for c in range(_D // _L):
                sl = pl.ds(c * _L, _L)
                df = (hblk[par, j, hr, sl] + rblk[par, j, rr, sl]
                      - tblk[par, j, tr, sl])
                sq = df * df
                acc = sq if acc is None else acc + sq
            ps.append(acc)
        for lvl in range(4):
            ps = [merge(ps[2 * j], ps[2 * j + 1], lvl)
                  for j in range(len(ps) // 2)]
        acc = ps[0]

        a = jnp.maximum(acc, jnp.float32(1e-30))
        i = lax.bitcast_convert_type(a, jnp.int32)
        i = jnp.int32(0x5F3759DF) - lax.shift_right_logical(
            i, jnp.ones((_L,), jnp.int32))
        y = lax.bitcast_convert_type(i, jnp.float32)
        half_a = jnp.float32(0.5) * a
        for _ in range(3):
            y = y * (jnp.float32(1.5) - half_a * y * y)
        outv[pl.ds(g * _L, _L)] = a * y
        return carry

    lax.fori_loop(0, _GROUPS, group_body, jnp.int32(0))
    pltpu.sync_copy(outv, out_hbm.at[pl.ds(base, _BPW)])


@functools.partial(
    pl.kernel,
    mesh=plsc.VectorSubcoreMesh(core_axis_name="c", subcore_axis_name="s"),
    out_type=jax.ShapeDtypeStruct((_B,), jnp.float32),
    compiler_params=pltpu.CompilerParams(use_tc_tiling_on_sc=True),
    scratch_types=[
        pltpu.VMEM((_BPW,), jnp.int32),
        pltpu.VMEM((_BPW,), jnp.int32),
        pltpu.VMEM((_BPW,), jnp.int32),
        pltpu.VMEM((2, _L, 8, _D), jnp.float32),
        pltpu.VMEM((2, _L, 8, _D), jnp.float32),
        pltpu.VMEM((2, _L, 8, _D), jnp.float32),
        pltpu.VMEM((_BPW,), jnp.float32),
        pltpu.SemaphoreType.DMA,
    ],
)
def _transe_sc(heads_hbm, rels_hbm, tails_hbm, ent, rel, out_hbm,
               hidx, ridx, tidx, hblk, rblk, tblk, outv, sem):
    _sc_body(heads_hbm, rels_hbm, tails_hbm, ent, rel, out_hbm,
             hidx, ridx, tidx, hblk, rblk, tblk, outv, sem)


def kernel(heads, relations, tails, entity_emb, relation_emb):
    h32 = heads.astype(jnp.int32)
    r32 = relations.astype(jnp.int32)
    t32 = tails.astype(jnp.int32)
    ent3 = entity_emb.reshape(125000, 8, _D)
    rel3 = relation_emb.reshape(125, 8, _D)
    return _transe_sc(h32, r32, t32, ent3, rel3)
